# baseline (speedup 1.0000x reference)
import functools

import jax
import jax.numpy as jnp
from jax import lax
from jax.experimental import pallas as pl
from jax.experimental.pallas import tpu as pltpu
from jax.experimental.pallas import tpu_sc as plsc

D_MODEL = 64
LANE = 128
PAIR = 2 * D_MODEL
TPAD = 129
RING = 4
WIN = 3

def _embed_sc(xj, xp, t2, n_pos, n_chunk):
    info = plsc.get_sparse_core_info()
    nc, ns = info.num_cores, info.num_subcores
    nw = nc * ns
    n_units = n_pos * n_chunk
    units_per_w = n_units // nw

    mesh = plsc.VectorSubcoreMesh(core_axis_name="c", subcore_axis_name="s")

    @functools.partial(
        pl.kernel,
        mesh=mesh,
        out_type=jax.ShapeDtypeStruct(
            (n_pos, 8, n_chunk, 8, LANE), jnp.float32
        ),
        scratch_types=[
            pltpu.VMEM((RING, LANE), jnp.int32),
            pltpu.VMEM((RING, LANE), jnp.int32),
            pltpu.VMEM((RING, LANE, PAIR), jnp.float32),
            pltpu.VMEM((RING, 8, 8, LANE), jnp.float32),
            pltpu.SemaphoreType.DMA((RING,)),
            pltpu.SemaphoreType.DMA((RING,)),
            pltpu.SemaphoreType.DMA((RING,)),
            pltpu.SemaphoreType.DMA((RING,)),
        ],
        compiler_params=pltpu.CompilerParams(
            use_tc_tiling_on_sc=True, needs_layout_passes=False
        ),
    )
    def k(xj_hbm, xp_hbm, t_hbm, out_hbm, idx_v, pb_v, garr, tbuf,
          isem, psem, gsem, osem):
        wid = lax.axis_index("s") * nc + lax.axis_index("c")
        wbase = wid * units_per_w

        def fire_idx(g, slot):
            pltpu.async_copy(
                xj_hbm.at[g // n_chunk, g % n_chunk], idx_v.at[slot],
                isem.at[slot],
            )
            pltpu.async_copy(
                xp_hbm.at[g // n_chunk, g % n_chunk], pb_v.at[slot],
                psem.at[slot],
            )

        def wait_idx(slot):
            pltpu.make_async_copy(
                xj_hbm.at[0, 0], idx_v.at[0], isem.at[slot]
            ).wait()
            pltpu.make_async_copy(
                xp_hbm.at[0, 0], pb_v.at[0], psem.at[slot]
            ).wait()

        def fire_gather(slot):
            pltpu.async_copy(
                t_hbm.at[idx_v.at[slot]], garr.at[slot], gsem.at[slot]
            )

        def wait_gather(slot):
            pltpu.make_async_copy(
                t_hbm.at[idx_v.at[0]], garr.at[0], gsem.at[slot]
            ).wait()

        def fire_store(g, slot):
            pltpu.async_copy(
                tbuf.at[slot],
                out_hbm.at[g // n_chunk, :, g % n_chunk],
                osem.at[slot],
            )

        def drain_store(slot):
            pltpu.make_async_copy(
                tbuf.at[0], out_hbm.at[0, :, 0], osem.at[slot]
            ).wait()

        iota16 = lax.iota(jnp.int32, 16)
        cvs = [iota16 + 16 * k for k in range(4)]
        trk = [cv >> 3 for cv in cvs]
        rk = [cv & 7 for cv in cvs]

        def transpose_unit(slot):
            src = garr.at[slot]
            dst = tbuf.at[slot]
            pb = pb_v.at[slot]

            @plsc.parallel_loop(0, LANE, 1, unroll=8)
            def _(l0):
                lvec = (iota16 + l0) & (LANE - 1)
                pv64 = plsc.load_gather(pb, [lvec]) * D_MODEL
                for kk in range(4):
                    colv = pv64 + cvs[kk]
                    v = plsc.load_gather(src, [lvec, colv])
                    plsc.store_scatter(dst, [trk[kk], rk[kk], lvec], v)

        for j in range(RING):
            fire_idx(wbase + j, j)

        def step(u, carry):
            slot = u % RING

            @pl.when(u < units_per_w)
            def _():
                @pl.when(u >= RING)
                def _():
                    drain_store(slot)

                wait_idx(slot)
                fire_gather(slot)

            @pl.when(u >= WIN)
            def _():
                v = u - WIN
                sv = v % RING
                wait_gather(sv)
                transpose_unit(sv)
                fire_store(wbase + v, sv)

                @pl.when(v + RING < units_per_w)
                def _():
                    fire_idx(wbase + v + RING, sv)

            return carry

        lax.fori_loop(0, units_per_w + WIN, step, 0)

        for v in range(units_per_w - RING, units_per_w):
            drain_store(v % RING)

    return k(xj, xp, t2)


def kernel(x, table):
    n_s, n_pos = x.shape
    n_chunk = n_s // LANE
    xt = jnp.swapaxes(x.astype(jnp.int32), 0, 1).reshape(n_pos, n_chunk, LANE)
    xj = xt >> 1
    xp = xt & 1
    t2 = table.reshape(-1, PAIR)
    o = _embed_sc(xj, xp, t2, n_pos, n_chunk)
    ot = o.transpose(2, 4, 0, 1, 3)
    return ot.reshape(n_s, n_pos, D_MODEL)


# speedup vs baseline: 1.3236x; 1.3236x over previous
import functools

import jax
import jax.numpy as jnp
from jax import lax
from jax.experimental import pallas as pl
from jax.experimental.pallas import tpu as pltpu
from jax.experimental.pallas import tpu_sc as plsc

D_MODEL = 64
LANE = 128
TPAD = 129
RING = 6
WIN = 4


def _embed_sc(xt, table, n_pos, n_chunk):
    info = plsc.get_sparse_core_info()
    nc, ns = info.num_cores, info.num_subcores
    nw = nc * ns
    n_units = n_pos * n_chunk
    units_per_w = n_units // nw

    mesh = plsc.VectorSubcoreMesh(core_axis_name="c", subcore_axis_name="s")

    @functools.partial(
        pl.kernel,
        mesh=mesh,
        out_type=jax.ShapeDtypeStruct(
            (n_pos, 8, n_chunk, 8, LANE), jnp.float32
        ),
        scratch_types=[
            pltpu.VMEM((RING, LANE), jnp.int32),
            pltpu.VMEM((RING, LANE, D_MODEL), jnp.float32),
            pltpu.VMEM((RING, 8, 8, TPAD), jnp.float32),
            pltpu.SemaphoreType.DMA((RING,)),
            pltpu.SemaphoreType.DMA((RING,)),
            pltpu.SemaphoreType.DMA((RING,)),
        ],
        compiler_params=pltpu.CompilerParams(
            use_tc_tiling_on_sc=False, needs_layout_passes=False
        ),
    )
    def k(x_hbm, table_hbm, out_hbm, idx_v, garr, tbuf, isem, gsem, osem):
        wid = lax.axis_index("s") * nc + lax.axis_index("c")
        wbase = wid * units_per_w

        def fire_idx(g, slot):
            pltpu.async_copy(
                x_hbm.at[g // n_chunk, g % n_chunk], idx_v.at[slot],
                isem.at[slot],
            )

        def wait_idx(slot):
            pltpu.make_async_copy(
                x_hbm.at[0, 0], idx_v.at[0], isem.at[slot]
            ).wait()

        def fire_gather(slot):
            pltpu.async_copy(
                table_hbm.at[idx_v.at[slot]], garr.at[slot], gsem.at[slot]
            )

        def wait_gather(slot):
            pltpu.make_async_copy(
                table_hbm.at[idx_v.at[0]], garr.at[0], gsem.at[slot]
            ).wait()

        def fire_store(g, slot):
            pltpu.async_copy(
                tbuf.at[slot, :, :, pl.ds(0, LANE)],
                out_hbm.at[g // n_chunk, :, g % n_chunk],
                osem.at[slot],
            )

        def drain_store(slot):
            pltpu.make_async_copy(
                tbuf.at[0, :, :, pl.ds(0, LANE)],
                out_hbm.at[0, :, 0],
                osem.at[slot],
            ).wait()

        iota16 = lax.iota(jnp.int32, 16)
        trk = [(iota16 + 16 * k) >> 3 for k in range(4)]
        rk = [(iota16 + 16 * k) & 7 for k in range(4)]

        def transpose_unit(slot):
            src = garr.at[slot]
            dst = tbuf.at[slot]

            @plsc.parallel_loop(0, LANE, 1, unroll=8)
            def _(l):
                lvec = jnp.full((16,), 0, jnp.int32) + l
                for k in range(4):
                    v = src[l, pl.ds(16 * k, 16)]
                    plsc.store_scatter(dst, [trk[k], rk[k], lvec], v)

        for j in range(RING):
            fire_idx(wbase + j, j)

        def step(u, carry):
            slot = u % RING

            @pl.when(u < units_per_w)
            def _():
                @pl.when(u >= RING)
                def _():
                    drain_store(slot)

                wait_idx(slot)
                fire_gather(slot)

            @pl.when(u >= WIN)
            def _():
                v = u - WIN
                sv = v % RING
                wait_gather(sv)
                transpose_unit(sv)
                fire_store(wbase + v, sv)

                @pl.when(v + RING < units_per_w)
                def _():
                    fire_idx(wbase + v + RING, sv)

            return carry

        lax.fori_loop(0, units_per_w + WIN, step, 0)

        for v in range(units_per_w - RING, units_per_w):
            drain_store(v % RING)

    return k(xt, table)


def kernel(x, table):
    n_s, n_pos = x.shape
    n_chunk = n_s // LANE
    xt = (
        jnp.swapaxes(x.astype(jnp.int32), 0, 1).reshape(n_pos, n_chunk, LANE)
        * 2
    )
    t2 = jnp.pad(table, ((0, 0), (0, D_MODEL))).reshape(-1, D_MODEL)
    o = _embed_sc(xt, t2, n_pos, n_chunk)
    ot = o.transpose(2, 4, 0, 1, 3)
    return ot.reshape(n_s, n_pos, D_MODEL)


# speedup vs baseline: 1.3253x; 1.0013x over previous
import functools

import jax
import jax.numpy as jnp
from jax import lax
from jax.experimental import pallas as pl
from jax.experimental.pallas import tpu as pltpu
from jax.experimental.pallas import tpu_sc as plsc

D_MODEL = 64
LANE = 128
TPAD = 129
RING = 7
WIN = 5


def _embed_sc(xt, table, n_pos, n_chunk):
    info = plsc.get_sparse_core_info()
    nc, ns = info.num_cores, info.num_subcores
    nw = nc * ns
    n_units = n_pos * n_chunk
    units_per_w = n_units // nw

    mesh = plsc.VectorSubcoreMesh(core_axis_name="c", subcore_axis_name="s")

    @functools.partial(
        pl.kernel,
        mesh=mesh,
        out_type=jax.ShapeDtypeStruct(
            (n_pos, 8, n_chunk, 8, LANE), jnp.float32
        ),
        scratch_types=[
            pltpu.VMEM((RING, LANE), jnp.int32),
            pltpu.VMEM((RING, LANE, D_MODEL), jnp.float32),
            pltpu.VMEM((RING, 8, 8, TPAD), jnp.float32),
            pltpu.SemaphoreType.DMA((RING,)),
            pltpu.SemaphoreType.DMA((RING,)),
            pltpu.SemaphoreType.DMA((RING,)),
        ],
        compiler_params=pltpu.CompilerParams(
            use_tc_tiling_on_sc=False, needs_layout_passes=False
        ),
    )
    def k(x_hbm, table_hbm, out_hbm, idx_v, garr, tbuf, isem, gsem, osem):
        wid = lax.axis_index("s") * nc + lax.axis_index("c")
        wbase = wid * units_per_w

        def fire_idx(g, slot):
            pltpu.async_copy(
                x_hbm.at[g // n_chunk, g % n_chunk], idx_v.at[slot],
                isem.at[slot],
            )

        def wait_idx(slot):
            pltpu.make_async_copy(
                x_hbm.at[0, 0], idx_v.at[0], isem.at[slot]
            ).wait()

        def fire_gather(slot):
            pltpu.async_copy(
                table_hbm.at[idx_v.at[slot]], garr.at[slot], gsem.at[slot]
            )

        def wait_gather(slot):
            pltpu.make_async_copy(
                table_hbm.at[idx_v.at[0]], garr.at[0], gsem.at[slot]
            ).wait()

        def fire_store(g, slot):
            pltpu.async_copy(
                tbuf.at[slot, :, :, pl.ds(0, LANE)],
                out_hbm.at[g // n_chunk, :, g % n_chunk],
                osem.at[slot],
            )

        def drain_store(slot):
            pltpu.make_async_copy(
                tbuf.at[0, :, :, pl.ds(0, LANE)],
                out_hbm.at[0, :, 0],
                osem.at[slot],
            ).wait()

        iota16 = lax.iota(jnp.int32, 16)
        trk = [(iota16 + 16 * k) >> 3 for k in range(4)]
        rk = [(iota16 + 16 * k) & 7 for k in range(4)]

        def transpose_unit(slot):
            src = garr.at[slot]
            dst = tbuf.at[slot]

            @plsc.parallel_loop(0, LANE, 1, unroll=8)
            def _(l):
                lvec = jnp.full((16,), 0, jnp.int32) + l
                for k in range(4):
                    v = src[l, pl.ds(16 * k, 16)]
                    plsc.store_scatter(dst, [trk[k], rk[k], lvec], v)

        for j in range(RING):
            fire_idx(wbase + j, j)

        def step(u, carry):
            slot = u % RING

            @pl.when(u < units_per_w)
            def _():
                @pl.when(u >= RING)
                def _():
                    drain_store(slot)

                wait_idx(slot)
                fire_gather(slot)

            @pl.when(u >= WIN)
            def _():
                v = u - WIN
                sv = v % RING
                wait_gather(sv)
                transpose_unit(sv)
                fire_store(wbase + v, sv)

                @pl.when(v + RING < units_per_w)
                def _():
                    fire_idx(wbase + v + RING, sv)

            return carry

        lax.fori_loop(0, units_per_w + WIN, step, 0)

        for v in range(units_per_w - RING, units_per_w):
            drain_store(v % RING)

    return k(xt, table)


def kernel(x, table):
    n_s, n_pos = x.shape
    n_chunk = n_s // LANE
    xt = (
        jnp.swapaxes(x.astype(jnp.int32), 0, 1).reshape(n_pos, n_chunk, LANE)
        * 2
    )
    t2 = jnp.pad(table, ((0, 0), (0, D_MODEL))).reshape(-1, D_MODEL)
    o = _embed_sc(xt, t2, n_pos, n_chunk)
    ot = o.transpose(2, 4, 0, 1, 3)
    return ot.reshape(n_s, n_pos, D_MODEL)
